# Initial kernel scaffold; baseline (speedup 1.0000x reference)
import jax
import jax.numpy as jnp
from jax.experimental import pallas as pl


def kernel(x, hyp_input, hyperedge_attr, TB_W, TB_b, HG_W, HG_b):
    raise NotImplementedError("write your pallas kernel here")


# speedup vs baseline: 11.6525x; 11.6525x over previous
import functools

import jax
import jax.numpy as jnp
from jax import lax
from jax.experimental import pallas as pl
from jax.experimental.pallas import tpu as pltpu
from jax.experimental.pallas import tpu_sc as plsc

_N = 10000
_P = 10240
_D = 128
_NNZ = 320000
_NC, _NS = 2, 16
_NW = _NC * _NS
_PER_W = _NNZ // _NW
_CW = 80
_NCH = _PER_W // _CW
_NG = _NCH // 5
_IDXROWS = _NNZ // _CW
_ROWS_T = _IDXROWS // _NW
_RPT = _P // _NS
_IB = 64

_mesh = plsc.VectorSubcoreMesh(
    core_axis_name="c", subcore_axis_name="s",
    num_cores=_NC, num_subcores=_NS)


@functools.partial(
    pl.kernel,
    out_type=jax.ShapeDtypeStruct((_NC * _P, _D), jnp.float32),
    mesh=_mesh,
    scratch_types=[
        pltpu.VMEM((_IB, _CW), jnp.int32),
        pltpu.VMEM((_IB, _CW), jnp.int32),
        pltpu.VMEM((_CW, _D), jnp.float32),
        pltpu.VMEM((_CW, _D), jnp.float32),
        pltpu.VMEM_SHARED((_P, _D), jnp.float32),
        pltpu.SemaphoreType.DMA,
        pltpu.SemaphoreType.DMA,
    ],
)
def _sc_stage(table, src, dst, out, srci, dsti, bufa, bufb, acc, sema, semb):
    c = lax.axis_index("c")
    s = lax.axis_index("s")
    wid = c * _NS + s

    z16 = jnp.zeros((16,), jnp.float32)

    def _zrow(r, carry):
        for j in range(_D // 16):
            bufa[r, pl.ds(j * 16, 16)] = z16
        return carry

    lax.fori_loop(0, _CW, _zrow, 0)
    for k in range(_RPT // _CW):
        pltpu.sync_copy(bufa, acc.at[pl.ds(s * _RPT + k * _CW, _CW)])
    plsc.subcore_barrier()

    def _scat(buf, i):
        pltpu.sync_copy(buf, acc.at[dsti.at[i]], add=True)

    for b0 in range(0, _ROWS_T, _IB):
        n = min(_IB, _ROWS_T - b0)
        pltpu.sync_copy(src.at[wid].at[pl.ds(b0, n)], srci.at[pl.ds(0, n)])
        pltpu.sync_copy(dst.at[wid].at[pl.ds(b0, n)], dsti.at[pl.ds(0, n)])

        def _body(o, carry):
            i = 2 * o
            da = pltpu.async_copy(table.at[srci.at[i]], bufa, sema)
            db = pltpu.async_copy(table.at[srci.at[i + 1]], bufb, semb)
            da.wait()
            _scat(bufa, i)
            db.wait()
            _scat(bufb, i + 1)
            return carry

        lax.fori_loop(0, n // 2, _body, 0)
        if n % 2:
            pltpu.async_copy(table.at[srci.at[n - 1]], bufa, sema).wait()
            _scat(bufa, n - 1)
    plsc.subcore_barrier()

    r0 = s * _RPT
    pltpu.sync_copy(acc.at[pl.ds(r0, _RPT)],
                    out.at[pl.ds(c * _P + r0, _RPT)])


@functools.partial(
    pl.kernel,
    out_type=jax.ShapeDtypeStruct((_NC * _P, _D), jnp.float32),
    mesh=_mesh,
    scratch_types=[
        pltpu.VMEM((_ROWS_T, _CW), jnp.int32),
        pltpu.VMEM((_CW, _D), jnp.float32),
        pltpu.VMEM((_CW, _D), jnp.float32),
        pltpu.VMEM_SHARED((_P, _D), jnp.float32),
    ],
)
def _sc_ones(dst, out, dsti, buf1, buf0, acc):
    c = lax.axis_index("c")
    s = lax.axis_index("s")
    wid = c * _NS + s

    pltpu.sync_copy(dst.at[wid], dsti)

    o16 = jnp.ones((16,), jnp.float32)
    z16 = jnp.zeros((16,), jnp.float32)

    def _fill(r, carry):
        for j in range(_D // 16):
            buf1[r, pl.ds(j * 16, 16)] = o16
            buf0[r, pl.ds(j * 16, 16)] = z16
        return carry

    lax.fori_loop(0, _CW, _fill, 0)
    for k in range(_RPT // _CW):
        pltpu.sync_copy(buf0, acc.at[pl.ds(s * _RPT + k * _CW, _CW)])
    plsc.subcore_barrier()

    def _sc(i, carry):
        pltpu.sync_copy(buf1, acc.at[dsti.at[i]], add=True)
        return carry

    lax.fori_loop(0, _ROWS_T, _sc, 0)
    plsc.subcore_barrier()

    r0 = s * _RPT
    pltpu.sync_copy(acc.at[pl.ds(r0, _RPT)],
                    out.at[pl.ds(c * _P + r0, _RPT)])


_BLK = 2048
_GRID = _P // _BLK
_BD = _BLK // _D

_HIGH = lax.Precision.HIGHEST


def _dot(a, b):
    return jnp.dot(a, b, preferred_element_type=jnp.float32, precision=_HIGH)


def _tb(x, w0, w1, w2, b0, b1, b2):
    t = _dot(x, w0) + b0
    u = _dot(x, w1) + b1
    v = _dot(x, w2) + b2
    return jnp.maximum(t + jax.nn.sigmoid(u) + v, 0.0)


def _wspec():
    return pl.BlockSpec((_D, _D), lambda i: (0, 0))


def _bspec():
    return pl.BlockSpec((1, _D), lambda i: (0, 0))


def _xspec():
    return pl.BlockSpec((_BLK, _D), lambda i: (i, 0))


def _qspec():
    return pl.BlockSpec((_NC, _BD, _D, _D), lambda i: (0, i, 0, 0))


def _dspec():
    return _qspec()


def _out_pd():
    return jax.ShapeDtypeStruct((_P, _D), jnp.float32)


def _tc_a_body(x, w0, w1, w2, b0, b1, b2, hw, o):
    h = _tb(x[...], w0[...], w1[...], w2[...], b0[...], b1[...], b2[...])
    o[...] = _dot(h, hw[...])


_tc_a = pl.pallas_call(
    _tc_a_body,
    out_shape=_out_pd(),
    grid=(_GRID,),
    in_specs=[_xspec()] + [_wspec()] * 3 + [_bspec()] * 3 + [_wspec()],
    out_specs=_xspec(),
)


def _inv_scale(dref):
    deg = dref[0] + dref[1]
    return jnp.where(deg > 0, 1.0 / deg, 0.0)


def _tc_b_body(p, d, o):
    s = (p[0] + p[1]) * _inv_scale(d)
    o[...] = s.reshape(_BLK, _D)


_tc_b = pl.pallas_call(
    _tc_b_body,
    out_shape=_out_pd(),
    grid=(_GRID,),
    in_specs=[_qspec(), _dspec()],
    out_specs=_xspec(),
)


def _tc_c_body(q, d, hgb, w10, w11, w12, b10, b11, b12,
               w20, w21, w22, b20, b21, b22, hw, o):
    g = ((q[0] + q[1]) * _inv_scale(d)).reshape(_BLK, _D) + hgb[...]
    h = _tb(g, w10[...], w11[...], w12[...], b10[...], b11[...], b12[...])
    h = _tb(h, w20[...], w21[...], w22[...], b20[...], b21[...], b22[...])
    o[...] = _dot(h, hw[...])


_tc_c = pl.pallas_call(
    _tc_c_body,
    out_shape=_out_pd(),
    grid=(_GRID,),
    in_specs=[_qspec(), _dspec(), _bspec()]
    + ([_wspec()] * 3 + [_bspec()] * 3) * 2 + [_wspec()],
    out_specs=_xspec(),
)


def _tc_d_body(q, d, hgb, w10, w11, w12, b10, b11, b12,
               w20, w21, w22, b20, b21, b22, o):
    g = ((q[0] + q[1]) * _inv_scale(d)).reshape(_BLK, _D) + hgb[...]
    h = _tb(g, w10[...], w11[...], w12[...], b10[...], b11[...], b12[...])
    o[...] = _tb(h, w20[...], w21[...], w22[...], b20[...], b21[...], b22[...])


_tc_d = pl.pallas_call(
    _tc_d_body,
    out_shape=_out_pd(),
    grid=(_GRID,),
    in_specs=[_qspec(), _dspec(), _bspec()]
    + ([_wspec()] * 3 + [_bspec()] * 3) * 2,
    out_specs=_xspec(),
)


def kernel(x, hyp_input, hyperedge_attr, TB_W, TB_b, HG_W, HG_b):
    del hyperedge_attr
    f32 = jnp.float32
    hyp = hyp_input.astype(jnp.int32)
    ni2 = hyp[0].reshape(_NW, _ROWS_T, _CW)
    ei2 = hyp[1].reshape(_NW, _ROWS_T, _CW)

    x_pad = jnp.concatenate(
        [x.astype(f32), jnp.zeros((_P - _N, _D), f32)], axis=0)

    tw = TB_W.astype(f32)
    tb = TB_b.astype(f32).reshape(5, 3, 1, _D)
    hw = HG_W.astype(f32)
    hb = HG_b.astype(f32).reshape(2, 1, _D)

    dn = _sc_ones(ni2).reshape(_NC, _P // _D, _D, _D)
    de = _sc_ones(ei2).reshape(_NC, _P // _D, _D, _D)

    def hg_pair(xw):
        e = _sc_stage(xw, ni2, ei2).reshape(_NC, _P // _D, _D, _D)
        ef = _tc_b(e, de)
        n = _sc_stage(ef, ei2, ni2).reshape(_NC, _P // _D, _D, _D)
        return n

    xw1 = _tc_a(x_pad, tw[0, 0], tw[0, 1], tw[0, 2],
                tb[0, 0], tb[0, 1], tb[0, 2], hw[0])
    n1 = hg_pair(xw1)
    xw2 = _tc_c(n1, dn, hb[0],
                tw[1, 0], tw[1, 1], tw[1, 2], tb[1, 0], tb[1, 1], tb[1, 2],
                tw[2, 0], tw[2, 1], tw[2, 2], tb[2, 0], tb[2, 1], tb[2, 2],
                hw[1])
    n2 = hg_pair(xw2)
    out = _tc_d(n2, dn, hb[1],
                tw[3, 0], tw[3, 1], tw[3, 2], tb[3, 0], tb[3, 1], tb[3, 2],
                tw[4, 0], tw[4, 1], tw[4, 2], tb[4, 0], tb[4, 1], tb[4, 2])
    return out[:_N]
